# baseline (speedup 1.0000x reference)
import jax
import jax.numpy as jnp
from jax.experimental import pallas as pl
from jax.experimental.pallas import tpu as pltpu

_K = 16


def _copy_body(x_hbm, o_hbm, sems):
    rows = x_hbm.shape[0] // _K

    def cp(j):
        return pltpu.make_async_copy(
            x_hbm.at[pl.ds(j * rows, rows), :],
            o_hbm.at[pl.ds(j * rows, rows), :],
            sems.at[j])

    for j in range(_K):
        cp(j).start()
    for j in range(_K):
        cp(j).wait()


def _copy_2d(x):
    b, t = x.shape
    return pl.pallas_call(
        _copy_body,
        out_shape=jax.ShapeDtypeStruct((b, t), x.dtype),
        in_specs=[pl.BlockSpec(memory_space=pl.ANY)],
        out_specs=pl.BlockSpec(memory_space=pl.ANY),
        scratch_shapes=[pltpu.SemaphoreType.DMA((_K,))],
    )(x)


def kernel(x):
    squeeze_batch = False
    if x.ndim == 1:
        x = x[None, :]
        squeeze_batch = True
    out = _copy_2d(x)
    if squeeze_batch:
        out = out[0]
    return out


# speedup vs baseline: 46.6513x; 46.6513x over previous
import jax
import jax.numpy as jnp
from jax.experimental import pallas as pl
from jax.experimental.pallas import tpu as pltpu

_K = 8


def _copy_body(x_hbm, o_hbm, buf, lsems, ssems):
    rows = buf.shape[1]

    def ld(j):
        return pltpu.make_async_copy(
            x_hbm.at[pl.ds(j * rows, rows), :], buf.at[j], lsems.at[j])

    def st(j):
        return pltpu.make_async_copy(
            buf.at[j], o_hbm.at[pl.ds(j * rows, rows), :], ssems.at[j])

    for j in range(_K):
        ld(j).start()
    for j in range(_K):
        ld(j).wait()
        st(j).start()
    for j in range(_K):
        st(j).wait()


def _copy_2d(x):
    b, t = x.shape
    rows = b // _K
    return pl.pallas_call(
        _copy_body,
        out_shape=jax.ShapeDtypeStruct((b, t), x.dtype),
        in_specs=[pl.BlockSpec(memory_space=pl.ANY)],
        out_specs=pl.BlockSpec(memory_space=pl.ANY),
        scratch_shapes=[
            pltpu.VMEM((_K, rows, t), jnp.float32),
            pltpu.SemaphoreType.DMA((_K,)),
            pltpu.SemaphoreType.DMA((_K,)),
        ],
    )(x)


def kernel(x):
    squeeze_batch = False
    if x.ndim == 1:
        x = x[None, :]
        squeeze_batch = True
    out = _copy_2d(x)
    if squeeze_batch:
        out = out[0]
    return out


# speedup vs baseline: 47.8317x; 1.0253x over previous
import jax
import jax.numpy as jnp
from jax.experimental import pallas as pl
from jax.experimental.pallas import tpu as pltpu


def _copy_block(x_ref, o_ref):
    o_ref[...] = x_ref[...]


def _copy_2d(x):
    b, t = x.shape
    rblk = 16
    if b % rblk != 0:
        rblk = b
    grid = b // rblk
    return pl.pallas_call(
        _copy_block,
        out_shape=jax.ShapeDtypeStruct((b, t), x.dtype),
        grid=(grid,),
        in_specs=[pl.BlockSpec((rblk, t), lambda i: (i, 0))],
        out_specs=pl.BlockSpec((rblk, t), lambda i: (i, 0)),
    )(x)


def kernel(x):
    squeeze_batch = False
    if x.ndim == 1:
        x = x[None, :]
        squeeze_batch = True
    out = _copy_2d(x)
    if squeeze_batch:
        out = out[0]
    return out
